# baseline (speedup 1.0000x reference)
import functools

import jax
import jax.numpy as jnp
from jax import lax
from jax.experimental import pallas as pl
from jax.experimental.pallas import tpu as pltpu
from jax.experimental.pallas import tpu_sc as plsc

B = 4096
S = 512
H = 768
L = 16
V = H // L
NW = 32
PP = S // NW
SB = 4
T = SB * PP
NCH = B // SB
EPS = 1e-12


def _ln_token(rows, bias_v, gamma_v, beta_v, q, t, p):
    acc = jnp.zeros((L,), jnp.float32)
    acc2 = jnp.zeros((L,), jnp.float32)
    for v in range(V):
        sl = pl.ds(v * L, L)
        x = rows[q, t, sl] + bias_v[p, sl]
        rows[q, t, sl] = x
        acc = acc + x
        acc2 = acc2 + x * x
    s1 = acc[0]
    s2 = acc2[0]
    for i in range(1, L):
        s1 = s1 + acc[i]
        s2 = s2 + acc2[i]
    mean = s1 * (1.0 / H)
    var = s2 * (1.0 / H) - mean * mean
    xs = var + EPS
    bi = lax.bitcast_convert_type(xs, jnp.int32)
    y = lax.bitcast_convert_type(jnp.int32(0x5F3759DF) - (bi >> 1), jnp.float32)
    for _ in range(3):
        y = y * (1.5 - 0.5 * xs * y * y)
    rv = jnp.full((L,), y, jnp.float32)
    mv = jnp.full((L,), mean, jnp.float32)
    for v in range(V):
        sl = pl.ds(v * L, L)
        x = rows[q, t, sl]
        rows[q, t, sl] = (x - mv) * rv * gamma_v[sl] + beta_v[sl]


def _sc_body(ids_hbm, wemb_hbm, bias_hbm, gamma_hbm, beta_hbm, out_hbm,
             bias_v, gamma_v, beta_v, idxf0, idxf1, rows,
             gsem0, gsem1, osem0, osem1):
    nc = 2
    wid = lax.axis_index("s") * nc + lax.axis_index("c")
    p0 = wid * PP
    idxfs = (idxf0, idxf1)
    gsems = (gsem0, gsem1)
    osems = (osem0, osem1)

    pltpu.sync_copy(bias_hbm.at[pl.ds(p0, PP)], bias_v)
    pltpu.sync_copy(gamma_hbm, gamma_v)
    pltpu.sync_copy(beta_hbm, beta_v)

    def load_and_fire(g, q):
        base = (wid * NCH + g) * T
        pltpu.sync_copy(ids_hbm.at[pl.ds(base, T)], idxfs[q])
        pltpu.make_async_copy(wemb_hbm.at[idxfs[q]], rows.at[q], gsems[q]).start()

    def gather_wait(q):
        pltpu.make_async_copy(wemb_hbm.at[idxfs[q]], rows.at[q], gsems[q]).wait()

    def out_copies(g, q):
        b0 = g * SB
        return [
            pltpu.make_async_copy(
                rows.at[q, pl.ds(i * PP, PP), :],
                out_hbm.at[b0 + i, pl.ds(p0, PP), :],
                osems[q])
            for i in range(SB)
        ]

    def out_fire(g, q):
        for c in out_copies(g, q):
            c.start()

    def out_wait(g, q):
        for c in out_copies(g, q):
            c.wait()

    load_and_fire(0, 0)

    def pair_body(gg, carry):
        for q in (0, 1):
            g = 2 * gg + q
            nq = 1 - q

            @pl.when(g >= 1)
            def _():
                out_wait(g - 1, nq)

            @pl.when(g + 1 < NCH)
            def _():
                load_and_fire(g + 1, nq)

            gather_wait(q)

            def tok_body(t, _c):
                p = lax.rem(t, PP)
                _ln_token(rows, bias_v, gamma_v, beta_v, q, t, p)
                return _c

            lax.fori_loop(0, T, tok_body, 0)
            out_fire(g, q)
        return carry

    lax.fori_loop(0, NCH // 2, pair_body, 0)
    out_wait(NCH - 1, 1)


@functools.partial(jax.jit, static_argnames=())
def _emb_ln(ids, wemb, bias, gamma, beta):
    mesh = plsc.VectorSubcoreMesh(core_axis_name="c", subcore_axis_name="s")
    f = functools.partial(
        pl.kernel,
        out_type=jax.ShapeDtypeStruct((B, S, H), jnp.float32),
        mesh=mesh,
        scratch_types=[
            pltpu.VMEM((PP, H), jnp.float32),
            pltpu.VMEM((H,), jnp.float32),
            pltpu.VMEM((H,), jnp.float32),
            pltpu.VMEM((T,), jnp.int32),
            pltpu.VMEM((T,), jnp.int32),
            pltpu.VMEM((2, T, H), jnp.float32),
            pltpu.SemaphoreType.DMA,
            pltpu.SemaphoreType.DMA,
            pltpu.SemaphoreType.DMA,
            pltpu.SemaphoreType.DMA,
        ],
    )(_sc_body)
    return f(ids, wemb, bias, gamma, beta)


def kernel(input_ids, word_embeddings, position_embeddings,
           token_type_embeddings, ln_gamma, ln_beta):
    ids = input_ids.astype(jnp.int32)
    ids_r = ids.reshape(NCH, SB, NW, PP).transpose(2, 0, 1, 3).reshape(-1)
    bias = position_embeddings + token_type_embeddings[0][None, :]
    return _emb_ln(ids_r, word_embeddings, bias, ln_gamma, ln_beta)
